# baseline (speedup 1.0000x reference)
import functools
import math

import jax
import jax.numpy as jnp
from jax import lax
from jax.experimental import pallas as pl
from jax.experimental.pallas import tpu as pltpu
from jax.experimental.pallas import tpu_sc as plsc

SIZE = 32000
N_ROWS = 4096
IGNORE_IDX = -100
SMOOTHING = 0.1
CONFIDENCE = 1.0 - SMOOTHING
IGN_COL = SIZE + IGNORE_IDX
EPS = SMOOTHING / (SIZE - 2)
_LOG_EPS = math.log(EPS)
ENT_A = (SIZE - 2) * EPS * _LOG_EPS + CONFIDENCE * math.log(CONFIDENCE)
ENT_B = (SIZE - 1) * EPS * _LOG_EPS

LANES = 16
NUM_WORKERS = 32
RPW = N_ROWS // NUM_WORKERS

NT = 0
NSC = N_ROWS - NT
RPW_SC = NSC // NUM_WORKERS
ACCS = 8
CHUNK = LANES * ACCS
ROW_ITERS = SIZE // CHUNK


def _sc_body(src1, trg_hbm, g_out, c_out, p_out,
             trg_v, idxg_v, idxc_v, gval_v, cval_v, buf0, buf1, psum_v,
             gsem, csem, sem0, sem1):
    wid = lax.axis_index("s") * 2 + lax.axis_index("c")
    base = wid * RPW
    pltpu.sync_copy(trg_hbm.at[pl.ds(base, RPW)], trg_v)
    iota = lax.iota(jnp.int32, LANES)
    for k in range(RPW // LANES):
        t = jnp.maximum(trg_v[pl.ds(k * LANES, LANES)], 0)
        rows = (base + k * LANES) + iota
        idxg_v[pl.ds(k * LANES, LANES)] = rows * SIZE + t
        idxc_v[pl.ds(k * LANES, LANES)] = rows * SIZE + IGN_COL
    gcp = pltpu.async_copy(src1.at[idxg_v], gval_v, gsem)
    ccp = pltpu.async_copy(src1.at[idxc_v], cval_v, csem)

    row0 = NT + wid * RPW_SC
    bufs = (buf0, buf1)
    sems = (sem0, sem1)
    cps = {0: pltpu.async_copy(src1.at[pl.ds(row0 * SIZE, SIZE)],
                               bufs[0], sems[0])}
    accs = tuple(jnp.zeros((LANES,), jnp.float32) for _ in range(ACCS))
    for r in range(RPW_SC):
        if r + 1 < RPW_SC:
            cps[r + 1] = pltpu.async_copy(
                src1.at[pl.ds((row0 + r + 1) * SIZE, SIZE)],
                bufs[(r + 1) % 2], sems[(r + 1) % 2])
        cps[r].wait()
        buf = bufs[r % 2]

        def body(k, a, buf=buf):
            o = k * CHUNK
            return tuple(aj + buf[pl.ds(o + j * LANES, LANES)]
                         for j, aj in enumerate(a))

        accs = lax.fori_loop(0, ROW_ITERS, body, accs)
    total = accs[0]
    for a in accs[1:]:
        total = total + a
    psum_v[...] = total
    gcp.wait()
    ccp.wait()
    pltpu.sync_copy(gval_v, g_out.at[pl.ds(base, RPW)])
    pltpu.sync_copy(cval_v, c_out.at[pl.ds(base, RPW)])
    pltpu.sync_copy(psum_v, p_out.at[pl.ds(wid * LANES, LANES)])


def _sc_part(src1, trg32):
    mesh = plsc.VectorSubcoreMesh(core_axis_name="c", subcore_axis_name="s")
    f = functools.partial(
        pl.kernel,
        mesh=mesh,
        out_type=[
            jax.ShapeDtypeStruct((N_ROWS,), jnp.float32),
            jax.ShapeDtypeStruct((N_ROWS,), jnp.float32),
            jax.ShapeDtypeStruct((NUM_WORKERS * LANES,), jnp.float32),
        ],
        scratch_types=[
            pltpu.VMEM((RPW,), jnp.int32),
            pltpu.VMEM((RPW,), jnp.int32),
            pltpu.VMEM((RPW,), jnp.int32),
            pltpu.VMEM((RPW,), jnp.float32),
            pltpu.VMEM((RPW,), jnp.float32),
            pltpu.VMEM((SIZE,), jnp.float32),
            pltpu.VMEM((SIZE,), jnp.float32),
            pltpu.VMEM((LANES,), jnp.float32),
            pltpu.SemaphoreType.DMA,
            pltpu.SemaphoreType.DMA,
            pltpu.SemaphoreType.DMA,
            pltpu.SemaphoreType.DMA,
        ],
    )(_sc_body)
    return f(src1, trg32)


BR = 128
GR = NT // BR


def _rowsum_body(src_ref, rs_ref):
    rs_ref[...] = jnp.sum(src_ref[...], axis=1, keepdims=True)


def _rowsum(src):
    return pl.pallas_call(
        _rowsum_body,
        grid=(GR,),
        in_specs=[pl.BlockSpec((BR, SIZE), lambda r: (r, 0))],
        out_specs=pl.BlockSpec((BR, 1), lambda r: (r, 0)),
        out_shape=jax.ShapeDtypeStruct((NT, 1), jnp.float32),
    )(src)


CR = 32
CC = N_ROWS // CR


def _combine_body(scp_ref, g_ref, cv_ref, trg_ref, out_ref):
    g = g_ref[...]
    cv = cv_ref[...]
    t = trg_ref[...]
    w_a = ENT_A + EPS * (g + cv) - CONFIDENCE * g
    w_b = ENT_B + EPS * cv
    w = jnp.where(t == IGN_COL, w_b, w_a)
    w = jnp.where(t == IGNORE_IDX, 0.0, w)
    total = jnp.sum(w) - EPS * jnp.sum(scp_ref[...])
    out_ref[...] = (total / N_ROWS).reshape(1, 1)


def _combine(scp, g, cv, trg32):
    return pl.pallas_call(
        _combine_body,
        out_shape=jax.ShapeDtypeStruct((1, 1), jnp.float32),
    )(scp.reshape(4, 128), g.reshape(CR, CC),
      cv.reshape(CR, CC), trg32.reshape(CR, CC))


def kernel(src, trg):
    trg32 = trg.astype(jnp.int32)
    g, cv, scp = _sc_part(src.reshape(-1), trg32)
    return _combine(scp, g, cv, trg32)[0, 0]


# speedup vs baseline: 1.0950x; 1.0950x over previous
import functools
import math

import jax
import jax.numpy as jnp
from jax import lax
from jax.experimental import pallas as pl
from jax.experimental.pallas import tpu as pltpu
from jax.experimental.pallas import tpu_sc as plsc

SIZE = 32000
N_ROWS = 4096
IGNORE_IDX = -100
SMOOTHING = 0.1
CONFIDENCE = 1.0 - SMOOTHING
IGN_COL = SIZE + IGNORE_IDX
EPS = SMOOTHING / (SIZE - 2)
_LOG_EPS = math.log(EPS)
ENT_A = (SIZE - 2) * EPS * _LOG_EPS + CONFIDENCE * math.log(CONFIDENCE)
ENT_B = (SIZE - 1) * EPS * _LOG_EPS

LANES = 16
NUM_WORKERS = 32
RPW = N_ROWS // NUM_WORKERS


def _sc_body(src2, trg_hbm, p_out, trg_v, grp_v, psum_v, gsem):
    wid = lax.axis_index("s") * 2 + lax.axis_index("c")
    base = wid * RPW
    pltpu.sync_copy(trg_hbm.at[pl.ds(base, RPW)], trg_v)
    cps = []
    for k in range(RPW // LANES):
        tv = jnp.maximum(trg_v[pl.ds(k * LANES, LANES)], 0)
        col0s = (tv >> 4) * LANES
        for j in range(LANES):
            r = k * LANES + j
            col0 = pl.multiple_of(col0s[j], LANES)
            cps.append(pltpu.async_copy(
                src2.at[base + r, pl.ds(col0, LANES)],
                grp_v.at[pl.ds(r * LANES, LANES)], gsem))
    for cp in cps:
        cp.wait()
    iota = lax.iota(jnp.int32, LANES)
    acc = jnp.zeros((LANES,), jnp.float32)
    for k in range(RPW // LANES):
        t = trg_v[pl.ds(k * LANES, LANES)]
        lanes = jnp.maximum(t, 0) & (LANES - 1)
        pos = (k * LANES * LANES) + iota * LANES + lanes
        keep = (t != IGN_COL) & (t != IGNORE_IDX)
        sel = jnp.zeros((LANES,), jnp.float32)
        for j in range(LANES):
            v = grp_v[pl.ds(pos[j] - j, LANES)]
            sel = jnp.where(iota == j, v, sel)
        acc = acc + jnp.where(keep, sel, 0.0)
    psum_v[...] = acc
    pltpu.sync_copy(psum_v, p_out.at[pl.ds(wid * LANES, LANES)])


def _sc_gather_sum(src, trg32):
    mesh = plsc.VectorSubcoreMesh(core_axis_name="c", subcore_axis_name="s")
    f = functools.partial(
        pl.kernel,
        mesh=mesh,
        compiler_params=pltpu.CompilerParams(use_tc_tiling_on_sc=False),
        out_type=jax.ShapeDtypeStruct((NUM_WORKERS * LANES,), jnp.float32),
        scratch_types=[
            pltpu.VMEM((RPW,), jnp.int32),
            pltpu.VMEM((RPW * LANES + LANES,), jnp.float32),
            pltpu.VMEM((LANES,), jnp.float32),
            pltpu.SemaphoreType.DMA,
        ],
    )(_sc_body)
    return f(src, trg32)


BR = 128
GR = N_ROWS // BR


def _rowsum_body(src_ref, rs_ref, cv_ref):
    rs_ref[...] = jnp.sum(src_ref[...], axis=1, keepdims=True)
    cv_ref[...] = src_ref[:, IGN_COL:IGN_COL + 1]


def _rowsum(src):
    return pl.pallas_call(
        _rowsum_body,
        grid=(GR,),
        in_specs=[pl.BlockSpec((BR, SIZE), lambda r: (r, 0))],
        out_specs=[
            pl.BlockSpec((BR, 1), lambda r: (r, 0)),
            pl.BlockSpec((BR, 1), lambda r: (r, 0)),
        ],
        out_shape=[
            jax.ShapeDtypeStruct((N_ROWS, 1), jnp.float32),
            jax.ShapeDtypeStruct((N_ROWS, 1), jnp.float32),
        ],
    )(src)


CR = 32
CC = N_ROWS // CR


def _combine_body(rs_ref, cv_ref, trg_ref, gsum_ref, out_ref):
    cv = cv_ref[...]
    t = trg_ref[...]
    w_a = ENT_A + EPS * cv
    w_b = ENT_B + EPS * cv
    w = jnp.where(t == IGN_COL, w_b, w_a)
    w = jnp.where(t == IGNORE_IDX, 0.0, w)
    total = (jnp.sum(w)
             + (EPS - CONFIDENCE) * jnp.sum(gsum_ref[...])
             - EPS * jnp.sum(rs_ref[...]))
    out_ref[...] = (total / N_ROWS).reshape(1, 1)


def _combine(rs, cv, trg32, gsum):
    return pl.pallas_call(
        _combine_body,
        out_shape=jax.ShapeDtypeStruct((1, 1), jnp.float32),
    )(rs.reshape(CR, CC), cv.reshape(CR, CC), trg32.reshape(CR, CC),
      gsum.reshape(4, 128))


def kernel(src, trg):
    trg32 = trg.astype(jnp.int32)
    gsum = _sc_gather_sum(src, trg32)
    rs, cv = _rowsum(src)
    return _combine(rs, cv, trg32, gsum)[0, 0]


# speedup vs baseline: 3.2008x; 2.9231x over previous
import functools
import math

import jax
import jax.numpy as jnp
from jax import lax
from jax.experimental import pallas as pl
from jax.experimental.pallas import tpu as pltpu
from jax.experimental.pallas import tpu_sc as plsc

SIZE = 32000
N_ROWS = 4096
IGNORE_IDX = -100
SMOOTHING = 0.1
CONFIDENCE = 1.0 - SMOOTHING
IGN_COL = SIZE + IGNORE_IDX
EPS = SMOOTHING / (SIZE - 2)
_LOG_EPS = math.log(EPS)
ENT_A = (SIZE - 2) * EPS * _LOG_EPS + CONFIDENCE * math.log(CONFIDENCE)
ENT_B = (SIZE - 1) * EPS * _LOG_EPS

LANES = 16
NUM_WORKERS = 32
RPW = N_ROWS // NUM_WORKERS


TILE_R = 8
TILE_C = 128


def _sc_body(src2, trg_hbm, p_out, trg_v, dst_v, psum_v, gsem):
    wid = lax.axis_index("s") * 2 + lax.axis_index("c")
    base = wid * RPW
    pltpu.sync_copy(trg_hbm.at[pl.ds(base, RPW)], trg_v)
    iota = lax.iota(jnp.int32, LANES)
    acc = jnp.zeros((LANES,), jnp.float32)
    for k in range(RPW // LANES):
        t_raw = trg_v[pl.ds(k * LANES, LANES)]
        t = jnp.maximum(t_raw, 0)
        c0s = (t >> 7) * TILE_C
        lj = t & (TILE_C - 1)
        cps = []
        for j in range(LANES):
            c0 = pl.multiple_of(c0s[j], TILE_C)
            row_t = base + k * LANES + (j & ~(TILE_R - 1))
            cps.append(pltpu.async_copy(
                src2.at[pl.ds(row_t, TILE_R), pl.ds(c0, TILE_C)],
                dst_v.at[pl.ds(j * TILE_R, TILE_R), :],
                gsem))
        for cp in cps:
            cp.wait()
        sel = jnp.zeros((LANES,), jnp.float32)
        for j in range(LANES):
            s = lj[j] - j
            v = dst_v[j * TILE_R + (j & (TILE_R - 1)), pl.ds(s, LANES)]
            sel = jnp.where(iota == j, v, sel)
        keep = (t_raw != IGN_COL) & (t_raw != IGNORE_IDX)
        acc = acc + jnp.where(keep, sel, 0.0)
    psum_v[...] = acc
    pltpu.sync_copy(psum_v, p_out.at[pl.ds(wid * LANES, LANES)])


def _sc_gather_sum(src, trg32):
    mesh = plsc.VectorSubcoreMesh(core_axis_name="c", subcore_axis_name="s")
    f = functools.partial(
        pl.kernel,
        mesh=mesh,
        out_type=jax.ShapeDtypeStruct((NUM_WORKERS * LANES,), jnp.float32),
        scratch_types=[
            pltpu.VMEM((RPW,), jnp.int32),
            pltpu.VMEM((LANES * TILE_R, TILE_C), jnp.float32),
            pltpu.VMEM((LANES,), jnp.float32),
            pltpu.SemaphoreType.DMA,
        ],
    )(_sc_body)
    return f(src, trg32)


BR = 128
GR = N_ROWS // BR


def _rowsum_body(src_ref, rs_ref, cv_ref):
    rs_ref[...] = jnp.sum(src_ref[...], axis=1, keepdims=True)
    cv_ref[...] = src_ref[:, IGN_COL:IGN_COL + 1]


def _rowsum(src):
    return pl.pallas_call(
        _rowsum_body,
        grid=(GR,),
        in_specs=[pl.BlockSpec((BR, SIZE), lambda r: (r, 0))],
        out_specs=[
            pl.BlockSpec((BR, 1), lambda r: (r, 0)),
            pl.BlockSpec((BR, 1), lambda r: (r, 0)),
        ],
        out_shape=[
            jax.ShapeDtypeStruct((N_ROWS, 1), jnp.float32),
            jax.ShapeDtypeStruct((N_ROWS, 1), jnp.float32),
        ],
    )(src)


CR = 32
CC = N_ROWS // CR


def _combine_body(rs_ref, cv_ref, trg_ref, gsum_ref, out_ref):
    cv = cv_ref[...]
    t = trg_ref[...]
    w_a = ENT_A + EPS * cv
    w_b = ENT_B + EPS * cv
    w = jnp.where(t == IGN_COL, w_b, w_a)
    w = jnp.where(t == IGNORE_IDX, 0.0, w)
    total = (jnp.sum(w)
             + (EPS - CONFIDENCE) * jnp.sum(gsum_ref[...])
             - EPS * jnp.sum(rs_ref[...]))
    out_ref[...] = (total / N_ROWS).reshape(1, 1)


def _combine(rs, cv, trg32, gsum):
    return pl.pallas_call(
        _combine_body,
        out_shape=jax.ShapeDtypeStruct((1, 1), jnp.float32),
    )(rs.reshape(CR, CC), cv.reshape(CR, CC), trg32.reshape(CR, CC),
      gsum.reshape(4, 128))


def kernel(src, trg):
    trg32 = trg.astype(jnp.int32)
    gsum = _sc_gather_sum(src, trg32)
    rs, cv = _rowsum(src)
    return _combine(rs, cv, trg32, gsum)[0, 0]
